# Initial kernel scaffold; baseline (speedup 1.0000x reference)
import jax
import jax.numpy as jnp
from jax.experimental import pallas as pl


def kernel(boxes, scores):
    raise NotImplementedError("write your pallas kernel here")


# speedup vs baseline: 179.6381x; 179.6381x over previous
import functools

import jax
import jax.numpy as jnp
from jax.experimental import pallas as pl
from jax.experimental.pallas import tpu as pltpu

_IOU_THR = 0.5
_SCORE_THR = 0.05
_MAX_OUT = 256
_B = 256


def _nms_body(rows_ref, cols_ref, out_ref, keepc_ref):
    npad = rows_ref.shape[1]
    nb = npad // _B

    iu = jax.lax.broadcasted_iota(jnp.int32, (_B, _B), 0)
    it = jax.lax.broadcasted_iota(jnp.int32, (_B, _B), 1)
    tri_strict = (iu < it).astype(jnp.float32)
    eye = (iu == it).astype(jnp.float32)
    lt_incl = (iu <= it).astype(jnp.float32)
    rrank = jax.lax.broadcasted_iota(
        jnp.int32, (_MAX_OUT, _B), 0).astype(jnp.float32) + 1.0

    out_ref[...] = jnp.zeros_like(out_ref)

    def row_to_col(v):
        return jnp.sum(eye * v, axis=1, keepdims=True)

    def iou_tile(cj, rk):
        jx1, jy1 = cj[:, 0:1], cj[:, 1:2]
        jx2, jy2 = cj[:, 2:3], cj[:, 3:4]
        kx1, ky1 = rk[0:1, :], rk[1:2, :]
        kx2, ky2 = rk[2:3, :], rk[3:4, :]
        xx1 = jnp.maximum(jx1, kx1)
        yy1 = jnp.maximum(jy1, ky1)
        xx2 = jnp.minimum(jx2, kx2)
        yy2 = jnp.minimum(jy2, ky2)
        inter = jnp.clip(xx2 - xx1, 0.0) * jnp.clip(yy2 - yy1, 0.0)
        areaj = (jx2 - jx1) * (jy2 - jy1)
        areak = (kx2 - kx1) * (ky2 - ky1)
        union = areaj + areak - inter
        return inter / jnp.maximum(union, 1e-9)

    def block_step(k, count):
        rk = rows_ref[:, pl.ds(k * _B, _B)]
        ck = cols_ref[pl.ds(k * _B, _B), :]
        s_blk = rk[4:5, :]

        def cross(j, sup):
            cj = cols_ref[pl.ds(j * _B, _B), :]
            keep_j = keepc_ref[pl.ds(j * _B, _B), :]
            m = iou_tile(cj, rk)
            contrib = jnp.max(
                jnp.where(m > _IOU_THR, keep_j, 0.0), axis=0, keepdims=True)
            return jnp.maximum(sup, contrib)

        sup_cross = jax.lax.fori_loop(
            0, k, cross, jnp.zeros((1, _B), jnp.float32))

        m_local = iou_tile(ck, rk)
        o_local = jnp.where(m_local > _IOU_THR, tri_strict, 0.0)
        alive = jnp.where(
            (s_blk > _SCORE_THR) & (sup_cross < 0.5), 1.0, 0.0)

        def fp_cond(carry):
            _, changed = carry
            return changed

        def fp_body(carry):
            keep, _ = carry
            kc = row_to_col(keep)
            sup = jnp.max(o_local * kc, axis=0, keepdims=True)
            new = alive * (1.0 - sup)
            return new, jnp.any(new != keep)

        keep_blk, _ = jax.lax.while_loop(
            fp_cond, fp_body, (alive, jnp.bool_(True)))

        keepc_ref[pl.ds(k * _B, _B), :] = row_to_col(keep_blk)

        local_cum = jax.lax.dot_general(
            keep_blk, lt_incl, (((1,), (0,)), ((), ())),
            preferred_element_type=jnp.float32)
        rank = local_cum + count
        sel = jnp.where((rank == rrank) & (keep_blk > 0.5), 1.0, 0.0)
        out_ref[...] += jax.lax.dot_general(
            sel, ck, (((1,), (0,)), ((), ())),
            preferred_element_type=jnp.float32)
        return count + jnp.sum(keep_blk)

    jax.lax.fori_loop(0, nb, block_step, jnp.float32(0.0))


@jax.jit
def kernel(boxes, scores):
    n = boxes.shape[0]
    npad = ((n + _B - 1) // _B) * _B
    order = jnp.argsort(-scores)
    b = boxes[order]
    s = scores[order]
    rows = jnp.zeros((8, npad), jnp.float32)
    rows = rows.at[0:4, :n].set(b.T)
    rows = rows.at[4, :n].set(s)
    cols = rows.T
    out8 = pl.pallas_call(
        _nms_body,
        out_shape=jax.ShapeDtypeStruct((_MAX_OUT, 8), jnp.float32),
        scratch_shapes=[pltpu.VMEM((npad, 1), jnp.float32)],
    )(rows, cols)
    return out8[:, :5]


# speedup vs baseline: 188.8612x; 1.0513x over previous
import functools

import jax
import jax.numpy as jnp
from jax import lax
from jax.experimental import pallas as pl
from jax.experimental.pallas import tpu as pltpu
from jax.experimental.pallas import tpu_sc as plsc

_IOU_THR = 0.5
_SCORE_THR = 0.05
_MAX_OUT = 256
_B = 256
_NPAD = 5120
_D = 16
_NC, _NS = 2, 16
_RPW = _NPAD // (_NC * _NS)


def _sc_gather_body(table_hbm, idx_hbm, out_hbm, idx_v, rows_v, sem):
    wid = lax.axis_index("s") * _NC + lax.axis_index("c")
    base = wid * _RPW
    pltpu.sync_copy(idx_hbm.at[pl.ds(base, _RPW)], idx_v)
    pltpu.async_copy(table_hbm.at[idx_v], rows_v, sem).wait()
    pltpu.sync_copy(rows_v, out_hbm.at[pl.ds(base, _RPW)])


_sc_gather = functools.partial(
    pl.kernel,
    mesh=plsc.VectorSubcoreMesh(core_axis_name="c", subcore_axis_name="s"),
    compiler_params=pltpu.CompilerParams(use_tc_tiling_on_sc=False),
    out_type=jax.ShapeDtypeStruct((_NPAD, _D), jnp.float32),
    scratch_types=[
        pltpu.VMEM((_RPW,), jnp.int32),
        pltpu.VMEM((_RPW, _D), jnp.float32),
        pltpu.SemaphoreType.DMA,
    ],
)(_sc_gather_body)


def _nms_body(rows_ref, cols_ref, out_ref, keepc_ref):
    npad = rows_ref.shape[1]
    nb = npad // _B

    iu = jax.lax.broadcasted_iota(jnp.int32, (_B, _B), 0)
    it = jax.lax.broadcasted_iota(jnp.int32, (_B, _B), 1)
    tri_strict = (iu < it).astype(jnp.float32)
    eye = (iu == it).astype(jnp.float32)
    lt_incl = (iu <= it).astype(jnp.float32)
    rrank = jax.lax.broadcasted_iota(
        jnp.int32, (_MAX_OUT, _B), 0).astype(jnp.float32) + 1.0

    out_ref[...] = jnp.zeros_like(out_ref)

    def row_to_col(v):
        return jnp.sum(eye * v, axis=1, keepdims=True)

    def iou_tile(cj, rk):
        jx1, jy1 = cj[:, 0:1], cj[:, 1:2]
        jx2, jy2 = cj[:, 2:3], cj[:, 3:4]
        kx1, ky1 = rk[0:1, :], rk[1:2, :]
        kx2, ky2 = rk[2:3, :], rk[3:4, :]
        xx1 = jnp.maximum(jx1, kx1)
        yy1 = jnp.maximum(jy1, ky1)
        xx2 = jnp.minimum(jx2, kx2)
        yy2 = jnp.minimum(jy2, ky2)
        inter = jnp.clip(xx2 - xx1, 0.0) * jnp.clip(yy2 - yy1, 0.0)
        areaj = (jx2 - jx1) * (jy2 - jy1)
        areak = (kx2 - kx1) * (ky2 - ky1)
        union = areaj + areak - inter
        return inter / jnp.maximum(union, 1e-9)

    def block_step(k, count):
        rk = rows_ref[:, pl.ds(k * _B, _B)]
        ck = cols_ref[pl.ds(k * _B, _B), :]
        s_blk = rk[4:5, :]

        def cross(j, sup):
            cj = cols_ref[pl.ds(j * _B, _B), :]
            keep_j = keepc_ref[pl.ds(j * _B, _B), :]
            m = iou_tile(cj, rk)
            contrib = jnp.max(
                jnp.where(m > _IOU_THR, keep_j, 0.0), axis=0, keepdims=True)
            return jnp.maximum(sup, contrib)

        sup_cross = jax.lax.fori_loop(
            0, k, cross, jnp.zeros((1, _B), jnp.float32))

        m_local = iou_tile(ck, rk)
        o_local = jnp.where(m_local > _IOU_THR, tri_strict, 0.0)
        alive = jnp.where(
            (s_blk > _SCORE_THR) & (sup_cross < 0.5), 1.0, 0.0)

        def fp_cond(carry):
            _, changed = carry
            return changed

        def fp_body(carry):
            keep, _ = carry
            kc = row_to_col(keep)
            sup = jnp.max(o_local * kc, axis=0, keepdims=True)
            new = alive * (1.0 - sup)
            return new, jnp.any(new != keep)

        keep_blk, _ = jax.lax.while_loop(
            fp_cond, fp_body, (alive, jnp.bool_(True)))

        keepc_ref[pl.ds(k * _B, _B), :] = row_to_col(keep_blk)

        local_cum = jax.lax.dot_general(
            keep_blk, lt_incl, (((1,), (0,)), ((), ())),
            preferred_element_type=jnp.float32)
        rank = local_cum + count
        sel = jnp.where((rank == rrank) & (keep_blk > 0.5), 1.0, 0.0)
        out_ref[...] += jax.lax.dot_general(
            sel, ck[:, :8], (((1,), (0,)), ((), ())),
            preferred_element_type=jnp.float32)
        return count + jnp.sum(keep_blk)

    jax.lax.fori_loop(0, nb, block_step, jnp.float32(0.0))


@jax.jit
def kernel(boxes, scores):
    n = boxes.shape[0]
    order = jnp.argsort(-scores).astype(jnp.int32)
    table = jnp.zeros((_NPAD, _D), jnp.float32)
    table = table.at[:n, 0:4].set(boxes)
    table = table.at[:n, 4].set(scores)
    idx = jnp.concatenate(
        [order, jnp.arange(n, _NPAD, dtype=jnp.int32)])
    cols = _sc_gather(table, idx)
    rows = cols.T
    out8 = pl.pallas_call(
        _nms_body,
        out_shape=jax.ShapeDtypeStruct((_MAX_OUT, 8), jnp.float32),
        scratch_shapes=[pltpu.VMEM((_NPAD, 1), jnp.float32)],
    )(rows, cols)
    return out8[:, :5]
